# baseline (speedup 1.0000x reference)
import functools

import jax
import jax.numpy as jnp
from jax import lax
from jax.experimental import pallas as pl
from jax.experimental.pallas import tpu as pltpu
from jax.experimental.pallas import tpu_sc as plsc

_IOU_THR = 0.5
_SCORE_THR = 0.05
_MAX_OUT = 256
_B = 512
_NPAD = 5120
_D = 16
_NC, _NS = 2, 16
_RPW = _NPAD // (_NC * _NS)


def _sc_gather_body(table_hbm, idx_hbm, out_hbm, idx_v, rows_v, sem):
    wid = lax.axis_index("s") * _NC + lax.axis_index("c")
    base = wid * _RPW
    pltpu.sync_copy(idx_hbm.at[pl.ds(base, _RPW)], idx_v)
    pltpu.async_copy(table_hbm.at[idx_v], rows_v, sem).wait()
    pltpu.sync_copy(rows_v, out_hbm.at[pl.ds(base, _RPW)])


@functools.cache
def _make_sc_gather():
    return functools.partial(
        pl.kernel,
        mesh=plsc.VectorSubcoreMesh(core_axis_name="c", subcore_axis_name="s"),
        compiler_params=pltpu.CompilerParams(use_tc_tiling_on_sc=False),
        out_type=jax.ShapeDtypeStruct((_NPAD, _D), jnp.float32),
        scratch_types=[
            pltpu.VMEM((_RPW,), jnp.int32),
            pltpu.VMEM((_RPW, _D), jnp.float32),
            pltpu.SemaphoreType.DMA,
        ],
    )(_sc_gather_body)


def _dot(a, b):
    return jax.lax.dot_general(
        a, b, (((1,), (0,)), ((), ())), preferred_element_type=jnp.float32)


_BIG = 1e30


def _nms_body(rows_ref, cols_ref, out_ref, supall_ref, hacol_ref, harow_ref):
    npad = rows_ref.shape[1]
    nb = npad // _B

    iu = jax.lax.broadcasted_iota(jnp.int32, (_B, _B), 0)
    it = jax.lax.broadcasted_iota(jnp.int32, (_B, _B), 1)
    tri_strict = (iu < it).astype(jnp.float32)
    eye = (iu == it).astype(jnp.float32)
    lt_incl = (iu <= it).astype(jnp.float32)
    rrank = jax.lax.broadcasted_iota(
        jnp.int32, (_MAX_OUT, _B), 0).astype(jnp.float32) + 1.0
    ones_r = jnp.ones((1, _B), jnp.float32)
    ones_c = jnp.ones((_B, 1), jnp.float32)

    out_ref[...] = jnp.zeros_like(out_ref)

    def row_to_col(v):
        return jnp.sum(eye * v, axis=1, keepdims=True)

    def prologue(k, _):
        ck = cols_ref[pl.ds(k * _B, _B), :]
        third = jnp.float32(1.0 / 3.0)
        hacol_ref[pl.ds(k * _B, _B), :] = (
            third * (ck[:, 2:3] - ck[:, 0:1]) * (ck[:, 3:4] - ck[:, 1:2]))
        rk = rows_ref[:, pl.ds(k * _B, _B)]
        harow_ref[0:1, pl.ds(k * _B, _B)] = (
            third * (rk[2:3, :] - rk[0:1, :]) * (rk[3:4, :] - rk[1:2, :]))
        return 0

    jax.lax.fori_loop(0, nb, prologue, 0)

    supall_ref[...] = jnp.zeros_like(supall_ref)

    def block_step(k, count):
        rk = rows_ref[:, pl.ds(k * _B, _B)]
        ck = cols_ref[pl.ds(k * _B, _B), :]
        s_blk = rk[4:5, :]
        sxx1 = jnp.broadcast_to(ck[:, 0:1], (_B, _B))
        syy1 = jnp.broadcast_to(ck[:, 1:2], (_B, _B))
        sxx2 = jnp.broadcast_to(ck[:, 2:3], (_B, _B))
        syy2 = jnp.broadcast_to(ck[:, 3:4], (_B, _B))

        def tile_cond(rt, hat3, pa_bb):
            iw = (jnp.minimum(sxx2, rt[2:3, :])
                  - jnp.maximum(sxx1, rt[0:1, :]))
            ih = (jnp.minimum(syy2, rt[3:4, :])
                  - jnp.maximum(syy1, rt[1:2, :]))
            inter = jnp.clip(iw, 0.0) * jnp.clip(ih, 0.0)
            return inter - hat3 > pa_bb

        ha3_col = hacol_ref[pl.ds(k * _B, _B), :]
        hak3_row = harow_ref[0:1, pl.ds(k * _B, _B)]
        cond_l = tile_cond(
            rk, hak3_row, jnp.broadcast_to(ha3_col, (_B, _B)))
        o_local = jnp.where(cond_l, tri_strict, 0.0)
        sup_cross = supall_ref[0:1, pl.ds(k * _B, _B)]
        alive = jnp.where(
            (s_blk > _SCORE_THR) & (sup_cross < 0.5), 1.0, 0.0)

        def fp_cond(carry):
            _, changed = carry
            return changed

        def fp_body(carry):
            keep, _ = carry
            kc = row_to_col(keep)
            sup = jnp.max(o_local * kc, axis=0, keepdims=True)
            new = alive * (1.0 - sup)
            return new, jnp.any(new != keep)

        keep_blk, _ = jax.lax.while_loop(
            fp_cond, fp_body, (alive, jnp.bool_(True)))

        keep_col = row_to_col(keep_blk)
        pa_bb = jnp.broadcast_to(
            jnp.where(keep_col > 0.5, ha3_col, _BIG), (_B, _B))

        def scatter(f, _):
            rf = rows_ref[:, pl.ds(f * _B, _B)]
            haf3 = harow_ref[0:1, pl.ds(f * _B, _B)]
            cond = tile_cond(rf, haf3, pa_bb)
            contrib = jnp.any(cond, axis=0, keepdims=True).astype(jnp.float32)
            supall_ref[0:1, pl.ds(f * _B, _B)] = jnp.maximum(
                supall_ref[0:1, pl.ds(f * _B, _B)], contrib)
            return 0

        jax.lax.fori_loop(k + 1, nb, scatter, 0)

        local_cum = _dot(keep_blk, lt_incl)
        rank = local_cum + count
        sel = jnp.where((rank == rrank) & (keep_blk > 0.5), 1.0, 0.0)
        out_ref[...] += _dot(sel, ck[:, :8])
        return count + jnp.sum(keep_blk)

    jax.lax.fori_loop(0, nb, block_step, jnp.float32(0.0))


@jax.jit
def kernel(boxes, scores):
    n = boxes.shape[0]
    order = jnp.argsort(-scores).astype(jnp.int32)
    table = jnp.zeros((_NPAD, _D), jnp.float32)
    table = table.at[:n, 0:4].set(boxes)
    table = table.at[:n, 4].set(scores)
    idx = jnp.concatenate(
        [order, jnp.arange(n, _NPAD, dtype=jnp.int32)])
    cols = _make_sc_gather()(table, idx)
    rows = cols.T
    out8 = pl.pallas_call(
        _nms_body,
        out_shape=jax.ShapeDtypeStruct((_MAX_OUT, 8), jnp.float32),
        scratch_shapes=[
            pltpu.VMEM((8, _NPAD), jnp.float32),
            pltpu.VMEM((_NPAD, 1), jnp.float32),
            pltpu.VMEM((8, _NPAD), jnp.float32),
        ],
    )(rows, cols)
    return out8[:, :5]


# speedup vs baseline: 1.0359x; 1.0359x over previous
import functools

import jax
import jax.numpy as jnp
from jax import lax
from jax.experimental import pallas as pl
from jax.experimental.pallas import tpu as pltpu
from jax.experimental.pallas import tpu_sc as plsc

_IOU_THR = 0.5
_SCORE_THR = 0.05
_MAX_OUT = 256
_B = 1024
_NPAD = 5120
_D = 16
_NC, _NS = 2, 16
_RPW = _NPAD // (_NC * _NS)


def _sc_gather_body(table_hbm, idx_hbm, out_hbm, idx_v, rows_v, sem):
    wid = lax.axis_index("s") * _NC + lax.axis_index("c")
    base = wid * _RPW
    pltpu.sync_copy(idx_hbm.at[pl.ds(base, _RPW)], idx_v)
    pltpu.async_copy(table_hbm.at[idx_v], rows_v, sem).wait()
    pltpu.sync_copy(rows_v, out_hbm.at[pl.ds(base, _RPW)])


@functools.cache
def _make_sc_gather():
    return functools.partial(
        pl.kernel,
        mesh=plsc.VectorSubcoreMesh(core_axis_name="c", subcore_axis_name="s"),
        compiler_params=pltpu.CompilerParams(use_tc_tiling_on_sc=False),
        out_type=jax.ShapeDtypeStruct((_NPAD, _D), jnp.float32),
        scratch_types=[
            pltpu.VMEM((_RPW,), jnp.int32),
            pltpu.VMEM((_RPW, _D), jnp.float32),
            pltpu.SemaphoreType.DMA,
        ],
    )(_sc_gather_body)


def _dot(a, b):
    return jax.lax.dot_general(
        a, b, (((1,), (0,)), ((), ())), preferred_element_type=jnp.float32)


_BIG = 1e30


def _nms_body(rows_ref, cols_ref, out_ref, supall_ref, hacol_ref, harow_ref):
    npad = rows_ref.shape[1]
    nb = npad // _B

    iu = jax.lax.broadcasted_iota(jnp.int32, (_B, _B), 0)
    it = jax.lax.broadcasted_iota(jnp.int32, (_B, _B), 1)
    tri_strict = (iu < it).astype(jnp.float32)
    eye = (iu == it).astype(jnp.float32)
    lt_incl = (iu <= it).astype(jnp.float32)
    rrank = jax.lax.broadcasted_iota(
        jnp.int32, (_MAX_OUT, _B), 0).astype(jnp.float32) + 1.0
    ones_r = jnp.ones((1, _B), jnp.float32)
    ones_c = jnp.ones((_B, 1), jnp.float32)

    out_ref[...] = jnp.zeros_like(out_ref)

    def row_to_col(v):
        return jnp.sum(eye * v, axis=1, keepdims=True)

    def prologue(k, _):
        ck = cols_ref[pl.ds(k * _B, _B), :]
        third = jnp.float32(1.0 / 3.0)
        hacol_ref[pl.ds(k * _B, _B), :] = (
            third * (ck[:, 2:3] - ck[:, 0:1]) * (ck[:, 3:4] - ck[:, 1:2]))
        rk = rows_ref[:, pl.ds(k * _B, _B)]
        harow_ref[0:1, pl.ds(k * _B, _B)] = (
            third * (rk[2:3, :] - rk[0:1, :]) * (rk[3:4, :] - rk[1:2, :]))
        return 0

    jax.lax.fori_loop(0, nb, prologue, 0)

    supall_ref[...] = jnp.zeros_like(supall_ref)

    def block_step(k, count):
        rk = rows_ref[:, pl.ds(k * _B, _B)]
        ck = cols_ref[pl.ds(k * _B, _B), :]
        s_blk = rk[4:5, :]
        sxx1 = jnp.broadcast_to(ck[:, 0:1], (_B, _B))
        syy1 = jnp.broadcast_to(ck[:, 1:2], (_B, _B))
        sxx2 = jnp.broadcast_to(ck[:, 2:3], (_B, _B))
        syy2 = jnp.broadcast_to(ck[:, 3:4], (_B, _B))

        def tile_cond(rt, hat3, pa_bb):
            iw = (jnp.minimum(sxx2, rt[2:3, :])
                  - jnp.maximum(sxx1, rt[0:1, :]))
            ih = (jnp.minimum(syy2, rt[3:4, :])
                  - jnp.maximum(syy1, rt[1:2, :]))
            inter = jnp.clip(iw, 0.0) * jnp.clip(ih, 0.0)
            return inter - hat3 > pa_bb

        ha3_col = hacol_ref[pl.ds(k * _B, _B), :]
        hak3_row = harow_ref[0:1, pl.ds(k * _B, _B)]
        cond_l = tile_cond(
            rk, hak3_row, jnp.broadcast_to(ha3_col, (_B, _B)))
        o_local = jnp.where(cond_l, tri_strict, 0.0)
        sup_cross = supall_ref[0:1, pl.ds(k * _B, _B)]
        alive = jnp.where(
            (s_blk > _SCORE_THR) & (sup_cross < 0.5), 1.0, 0.0)

        def fp_cond(carry):
            _, changed = carry
            return changed

        def fp_body(carry):
            keep, _ = carry
            kc = row_to_col(keep)
            sup = jnp.max(o_local * kc, axis=0, keepdims=True)
            new = alive * (1.0 - sup)
            return new, jnp.any(new != keep)

        keep_blk, _ = jax.lax.while_loop(
            fp_cond, fp_body, (alive, jnp.bool_(True)))

        keep_col = row_to_col(keep_blk)
        pa_bb = jnp.broadcast_to(
            jnp.where(keep_col > 0.5, ha3_col, _BIG), (_B, _B))

        def scatter(f, _):
            rf = rows_ref[:, pl.ds(f * _B, _B)]
            haf3 = harow_ref[0:1, pl.ds(f * _B, _B)]
            cond = tile_cond(rf, haf3, pa_bb)
            contrib = jnp.any(cond, axis=0, keepdims=True).astype(jnp.float32)
            supall_ref[0:1, pl.ds(f * _B, _B)] = jnp.maximum(
                supall_ref[0:1, pl.ds(f * _B, _B)], contrib)
            return 0

        jax.lax.fori_loop(k + 1, nb, scatter, 0)

        local_cum = _dot(keep_blk, lt_incl)
        rank = local_cum + count
        sel = jnp.where((rank == rrank) & (keep_blk > 0.5), 1.0, 0.0)
        out_ref[...] += _dot(sel, ck[:, :8])
        return count + jnp.sum(keep_blk)

    jax.lax.fori_loop(0, nb, block_step, jnp.float32(0.0))


@jax.jit
def kernel(boxes, scores):
    n = boxes.shape[0]
    order = jnp.argsort(-scores).astype(jnp.int32)
    table = jnp.zeros((_NPAD, _D), jnp.float32)
    table = table.at[:n, 0:4].set(boxes)
    table = table.at[:n, 4].set(scores)
    idx = jnp.concatenate(
        [order, jnp.arange(n, _NPAD, dtype=jnp.int32)])
    cols = _make_sc_gather()(table, idx)
    rows = cols.T
    out8 = pl.pallas_call(
        _nms_body,
        out_shape=jax.ShapeDtypeStruct((_MAX_OUT, 8), jnp.float32),
        scratch_shapes=[
            pltpu.VMEM((8, _NPAD), jnp.float32),
            pltpu.VMEM((_NPAD, 1), jnp.float32),
            pltpu.VMEM((8, _NPAD), jnp.float32),
        ],
    )(rows, cols)
    return out8[:, :5]


# speedup vs baseline: 1.0363x; 1.0004x over previous
import functools

import jax
import jax.numpy as jnp
from jax import lax
from jax.experimental import pallas as pl
from jax.experimental.pallas import tpu as pltpu
from jax.experimental.pallas import tpu_sc as plsc

_IOU_THR = 0.5
_SCORE_THR = 0.05
_MAX_OUT = 256
_B = 1024
_NPAD = 5120
_D = 16
_NC, _NS = 2, 16
_RPW = _NPAD // (_NC * _NS)


def _sc_gather_body(table_hbm, idx_hbm, out_hbm, idx_v, rows_v, sem):
    wid = lax.axis_index("s") * _NC + lax.axis_index("c")
    base = wid * _RPW
    pltpu.sync_copy(idx_hbm.at[pl.ds(base, _RPW)], idx_v)
    pltpu.async_copy(table_hbm.at[idx_v], rows_v, sem).wait()
    pltpu.sync_copy(rows_v, out_hbm.at[pl.ds(base, _RPW)])


@functools.cache
def _make_sc_gather():
    return functools.partial(
        pl.kernel,
        mesh=plsc.VectorSubcoreMesh(core_axis_name="c", subcore_axis_name="s"),
        compiler_params=pltpu.CompilerParams(use_tc_tiling_on_sc=False),
        out_type=jax.ShapeDtypeStruct((_NPAD, _D), jnp.float32),
        scratch_types=[
            pltpu.VMEM((_RPW,), jnp.int32),
            pltpu.VMEM((_RPW, _D), jnp.float32),
            pltpu.SemaphoreType.DMA,
        ],
    )(_sc_gather_body)


def _dot(a, b):
    return jax.lax.dot_general(
        a, b, (((1,), (0,)), ((), ())), preferred_element_type=jnp.float32)


_BIG = 1e30


def _nms_body(rows_ref, cols_ref, out_ref, supall_ref, hacol_ref, harow_ref):
    npad = rows_ref.shape[1]
    nb = npad // _B

    iu = jax.lax.broadcasted_iota(jnp.int32, (_B, _B), 0)
    it = jax.lax.broadcasted_iota(jnp.int32, (_B, _B), 1)
    tri_strict = (iu < it).astype(jnp.float32)
    eye = (iu == it).astype(jnp.float32)
    lt_incl = (iu <= it).astype(jnp.float32)
    rrank = jax.lax.broadcasted_iota(
        jnp.int32, (_MAX_OUT, _B), 0).astype(jnp.float32) + 1.0

    out_ref[...] = jnp.zeros_like(out_ref)

    def row_to_col(v):
        return jnp.sum(eye * v, axis=1, keepdims=True)

    def prologue(k, _):
        ck = cols_ref[pl.ds(k * _B, _B), :]
        third = jnp.float32(1.0 / 3.0)
        hacol_ref[pl.ds(k * _B, _B), :] = (
            third * (ck[:, 2:3] - ck[:, 0:1]) * (ck[:, 3:4] - ck[:, 1:2]))
        rk = rows_ref[:, pl.ds(k * _B, _B)]
        harow_ref[0:1, pl.ds(k * _B, _B)] = (
            third * (rk[2:3, :] - rk[0:1, :]) * (rk[3:4, :] - rk[1:2, :]))
        return 0

    jax.lax.fori_loop(0, nb, prologue, 0)

    supall_ref[...] = jnp.zeros_like(supall_ref)

    def block_step(k, count):
        rk = rows_ref[:, pl.ds(k * _B, _B)]
        ck = cols_ref[pl.ds(k * _B, _B), :]
        s_blk = rk[4:5, :]
        sxx1 = jnp.broadcast_to(ck[:, 0:1], (_B, _B))
        syy1 = jnp.broadcast_to(ck[:, 1:2], (_B, _B))
        sxx2 = jnp.broadcast_to(ck[:, 2:3], (_B, _B))
        syy2 = jnp.broadcast_to(ck[:, 3:4], (_B, _B))

        def tile_cond(rt, hat3, pa_bb):
            iw = (jnp.minimum(sxx2, rt[2:3, :])
                  - jnp.maximum(sxx1, rt[0:1, :]))
            ih = (jnp.minimum(syy2, rt[3:4, :])
                  - jnp.maximum(syy1, rt[1:2, :]))
            inter = jnp.clip(iw, 0.0) * jnp.clip(ih, 0.0)
            return inter - hat3 > pa_bb

        ha3_col = hacol_ref[pl.ds(k * _B, _B), :]
        hak3_row = harow_ref[0:1, pl.ds(k * _B, _B)]
        cond_l = tile_cond(
            rk, hak3_row, jnp.broadcast_to(ha3_col, (_B, _B)))
        o_local = jnp.where(cond_l, tri_strict, 0.0)
        sup_cross = supall_ref[0:1, pl.ds(k * _B, _B)]
        alive = jnp.where(
            (s_blk > _SCORE_THR) & (sup_cross < 0.5), 1.0, 0.0)

        def fp_cond(carry):
            _, changed = carry
            return changed

        def fp_body(carry):
            keep, _ = carry
            kc = row_to_col(keep)
            sup = jnp.max(o_local * kc, axis=0, keepdims=True)
            new = alive * (1.0 - sup)
            return new, jnp.any(new != keep)

        keep_blk, _ = jax.lax.while_loop(
            fp_cond, fp_body, (alive, jnp.bool_(True)))

        keep_col = row_to_col(keep_blk)
        pa_bb = jnp.broadcast_to(
            jnp.where(keep_col > 0.5, ha3_col, _BIG), (_B, _B))

        def scatter(f, _):
            rf = rows_ref[:, pl.ds(f * _B, _B)]
            haf3 = harow_ref[0:1, pl.ds(f * _B, _B)]
            cond = tile_cond(rf, haf3, pa_bb)
            contrib = jnp.any(cond, axis=0, keepdims=True).astype(jnp.float32)
            supall_ref[0:1, pl.ds(f * _B, _B)] = jnp.maximum(
                supall_ref[0:1, pl.ds(f * _B, _B)], contrib)
            return 0

        jax.lax.fori_loop(k + 1, nb, scatter, 0)

        local_cum = _dot(keep_blk, lt_incl)
        rank = local_cum + count
        sel = jnp.where((rank == rrank) & (keep_blk > 0.5), 1.0, 0.0)
        out_ref[...] += _dot(sel, ck[:, :8])
        return count + jnp.sum(keep_blk)

    jax.lax.fori_loop(0, nb, block_step, jnp.float32(0.0))


@jax.jit
def kernel(boxes, scores):
    n = boxes.shape[0]
    order = jnp.argsort(-scores).astype(jnp.int32)
    table = jnp.zeros((_NPAD, _D), jnp.float32)
    table = table.at[:n, 0:4].set(boxes)
    table = table.at[:n, 4].set(scores)
    idx = jnp.concatenate(
        [order, jnp.arange(n, _NPAD, dtype=jnp.int32)])
    cols = _make_sc_gather()(table, idx)
    rows = cols.T
    out8 = pl.pallas_call(
        _nms_body,
        out_shape=jax.ShapeDtypeStruct((_MAX_OUT, 8), jnp.float32),
        scratch_shapes=[
            pltpu.VMEM((8, _NPAD), jnp.float32),
            pltpu.VMEM((_NPAD, 1), jnp.float32),
            pltpu.VMEM((8, _NPAD), jnp.float32),
        ],
    )(rows, cols)
    return out8[:, :5]
